# baseline (speedup 1.0000x reference)
import functools

import jax
import jax.numpy as jnp
from jax import lax
from jax.experimental import pallas as pl
from jax.experimental.pallas import tpu as pltpu
from jax.experimental.pallas import tpu_sc as plsc

B = 16384
K = 16
LANES = 16
NC = 2
NS = 16
NW = NC * NS
BPW = B // NW
CHUNK = 128
NCHUNK = BPW // CHUNK


def _mf_body(uid_hbm, iid_hbm, user_hbm, item_hbm, out_hbm,
             idx_u, idx_i, u_rows, v_rows, p_flat, out_v, sem):
    wid = lax.axis_index("s") * NC + lax.axis_index("c")

    pltpu.sync_copy(uid_hbm.at[wid], idx_u)
    pltpu.sync_copy(iid_hbm.at[wid], idx_i)

    copies = []
    for j in range(NCHUNK):
        copies.append(pltpu.async_copy(
            user_hbm.at[idx_u.at[j]], u_rows.at[pl.ds(j * CHUNK, CHUNK)], sem))
        copies.append(pltpu.async_copy(
            item_hbm.at[idx_i.at[j]], v_rows.at[pl.ds(j * CHUNK, CHUNK)], sem))
    for c in copies:
        c.wait()

    iota = lax.iota(jnp.int32, LANES)

    def prod_body(r, carry):
        p_flat[pl.ds(r * K, K)] = u_rows[r, :] * v_rows[r, :]
        return carry

    lax.fori_loop(0, BPW, prod_body, 0)

    def red_body(g, carry):
        base = g * (LANES * K) + iota * K
        acc = jnp.zeros((LANES,), jnp.float32)
        for k in range(K):
            acc = acc + plsc.load_gather(p_flat, [base + k])
        out_v[pl.ds(g * LANES, LANES)] = acc
        return carry

    lax.fori_loop(0, BPW // LANES, red_body, 0)

    pltpu.sync_copy(out_v, out_hbm.at[wid])


@jax.jit
def kernel(uid, iid, user_mat, item_mat):
    uid3 = uid.astype(jnp.int32).reshape(NW, NCHUNK, CHUNK)
    iid3 = iid.astype(jnp.int32).reshape(NW, NCHUNK, CHUNK)

    run = pl.kernel(
        _mf_body,
        out_type=jax.ShapeDtypeStruct((NW, BPW), jnp.float32),
        mesh=plsc.VectorSubcoreMesh(core_axis_name="c", subcore_axis_name="s"),
        compiler_params=pltpu.CompilerParams(
            needs_layout_passes=False, use_tc_tiling_on_sc=False),
        scratch_types=[
            pltpu.VMEM((NCHUNK, CHUNK), jnp.int32),
            pltpu.VMEM((NCHUNK, CHUNK), jnp.int32),
            pltpu.VMEM((BPW, K), jnp.float32),
            pltpu.VMEM((BPW, K), jnp.float32),
            pltpu.VMEM((BPW * K,), jnp.float32),
            pltpu.VMEM((BPW,), jnp.float32),
            pltpu.SemaphoreType.DMA,
        ],
    )
    out = run(uid3, iid3, user_mat, item_mat)
    return out.reshape(B)
